# baseline (speedup 1.0000x reference)
import functools

import jax
import jax.numpy as jnp
import numpy as np
from jax import lax
from jax.experimental import pallas as pl
from jax.experimental.pallas import tpu as pltpu
from jax.experimental.pallas import tpu_sc as plsc

N = 10000
E = 320000
B = 4
D = 128
H = 128
EMB = 64
ATT = 64

NC = 2
NS = 16
NW = NC * NS
EP_W = E // NW
K = 80
NCHUNK = 133
EPAD = NW * NCHUNK * K
ZROWS = 125
RP_S = N // NS

_mesh = plsc.VectorSubcoreMesh(core_axis_name="c", subcore_axis_name="s")


@functools.partial(
    pl.kernel,
    out_type=jax.ShapeDtypeStruct((NW, N), jnp.float32),
    mesh=_mesh,
    scratch_types=[
        pltpu.VMEM((EP_W,), jnp.int32),
        pltpu.VMEM((EP_W,), jnp.float32),
        pltpu.VMEM((N,), jnp.float32),
    ],
    compiler_params=pltpu.CompilerParams(needs_layout_passes=False, use_tc_tiling_on_sc=False),
)
def _deg_kernel(dst_hbm, ew_hbm, degs_hbm, dst_v, ew_v, deg_v):
    wid = lax.axis_index("s") * NC + lax.axis_index("c")
    base = wid * EP_W

    def zero(i, carry):
        deg_v[pl.ds(i * 16, 16)] = jnp.zeros((16,), jnp.float32)
        return carry

    lax.fori_loop(0, N // 16, zero, 0)

    pltpu.sync_copy(dst_hbm.at[pl.ds(base, EP_W)], dst_v)
    pltpu.sync_copy(ew_hbm.at[pl.ds(base, EP_W)], ew_v)

    def body(j, carry):
        d16 = dst_v[pl.ds(j * 16, 16)]
        w16 = ew_v[pl.ds(j * 16, 16)]
        plsc.addupdate_scatter(deg_v, [d16], w16)
        return carry

    lax.fori_loop(0, EP_W // 16, body, 0)
    pltpu.sync_copy(deg_v, degs_hbm.at[wid])


@functools.partial(
    pl.kernel,
    out_type=jax.ShapeDtypeStruct((NC, N, H), jnp.float32),
    mesh=_mesh,
    scratch_types=(
        [pltpu.VMEM((3, K), jnp.int32)] * 8
        + [pltpu.VMEM((K, H // 2), jnp.int32)] * 4
        + [pltpu.VMEM((K, H), jnp.float32)] * 2
        + [pltpu.VMEM_SHARED((N, H), jnp.float32)]
        + [pltpu.SemaphoreType.DMA] * 14
    ),
    compiler_params=pltpu.CompilerParams(needs_layout_passes=False, use_tc_tiling_on_sc=False),
)
def _agg_kernel(epk_hbm, ys_hbm, parts_hbm, *refs):
    ib = refs[0:8]
    rows = refs[8:12]
    srows = refs[12:14]
    acc_sh = refs[14]
    isem = refs[15:23]
    gsem = refs[23:27]
    ssem = refs[27:29]

    c = lax.axis_index("c")
    s = lax.axis_index("s")
    wid = s * NC + c
    ibase = wid * NCHUNK

    def zero(i, carry):
        for v in range(H // 16):
            srows[0][i, pl.ds(v * 16, 16)] = jnp.zeros((16,), jnp.float32)
        return carry

    lax.fori_loop(0, K, zero, 0)
    for k in range(RP_S // K):
        pltpu.sync_copy(srows[0], acc_sh.at[pl.ds(s * RP_S + k * K, K)])
    rem = RP_S % K
    if rem:
        pltpu.sync_copy(srows[0].at[pl.ds(0, rem)],
                        acc_sh.at[pl.ds(s * RP_S + (RP_S // K) * K, rem)])
    plsc.subcore_barrier()

    def idx_start(cix, p):
        pltpu.async_copy(epk_hbm.at[ibase + cix], ib[p], isem[p])

    def idx_wait(p):
        pltpu.make_async_copy(epk_hbm.at[0], ib[p], isem[p]).wait()

    def gather_start(p, b):
        pltpu.async_copy(ys_hbm.at[ib[p].at[0]], rows[b], gsem[b])

    def gather_wait(p, b):
        pltpu.make_async_copy(ys_hbm.at[ib[p].at[0]], rows[b],
                              gsem[b]).wait()

    def scale(p, b, e):
        two16 = jnp.full((16,), 2, jnp.int32)
        msk = jnp.full((16,), -65536, jnp.int32)

        @plsc.parallel_loop(0, K, unroll=8)
        def body(j):
            idx16 = jnp.zeros((16,), jnp.int32) + j
            raw = plsc.load_gather(ib[p], [two16, idx16])
            bc = plsc.bitcast(raw, jnp.float32)
            for v in range(H // 32):
                pk = rows[b][j, pl.ds(v * 16, 16)]
                lo = plsc.bitcast(pk << 16, jnp.float32)
                hi = plsc.bitcast(pk & msk, jnp.float32)
                srows[e][j, pl.ds(v * 32, 16)] = lo * bc
                srows[e][j, pl.ds(v * 32 + 16, 16)] = hi * bc

    def scatter_start(p, e):
        pltpu.async_copy(srows[e], acc_sh.at[ib[p].at[1]], ssem[e], add=True)

    def scatter_wait(p, e):
        pltpu.make_async_copy(srows[e], acc_sh.at[ib[p].at[1]],
                              ssem[e]).wait()

    def step(cix, m, first=False, last=False, more_idx=True):
        p, b, e = m % 8, m % 4, m % 2
        p2, b2 = (m + 2) % 8, (m + 2) % 4
        if not first:
            scatter_wait((m - 2) % 8, (m - 2) % 2)
        if not last:
            idx_wait(p2)
            gather_start(p2, b2)
        if more_idx:
            idx_start(cix + 5, (m + 5) % 8)
        gather_wait(p, b)
        scale(p, b, e)
        scatter_start(p, e)

    for cix in range(5):
        idx_start(cix, cix)
    idx_wait(0)
    gather_start(0, 0)
    idx_wait(1)
    gather_start(1, 1)
    step(0, 0, first=True)
    step(1, 1, first=True)
    step(2, 2)
    step(3, 3)
    step(4, 4)

    def grp(kk, carry):
        base = 5 + 8 * kk
        for i in range(8):
            step(base + i, 5 + i)
        return carry

    lax.fori_loop(0, (NCHUNK - 5) // 8 - 1, grp, 0)
    for cix in range(NCHUNK - 8, NCHUNK):
        step(cix, cix, last=(cix + 2 >= NCHUNK),
             more_idx=(cix + 5 < NCHUNK))
    scatter_wait((NCHUNK - 2) % 8, (NCHUNK - 2) % 2)
    scatter_wait((NCHUNK - 1) % 8, (NCHUNK - 1) % 2)

    plsc.subcore_barrier()
    for k in range(RP_S // ZROWS):
        off = s * RP_S + k * ZROWS
        pltpu.sync_copy(acc_sh.at[pl.ds(off, ZROWS)],
                        parts_hbm.at[c, pl.ds(off, ZROWS)])


def _tcA_body(x_ref, w1_ref, degs_ref, ys_ref, dinv_ref):
    deg = jnp.sum(degs_ref[...], axis=0) + 1.0
    dinv = lax.rsqrt(deg)
    xw = lax.dot_general(x_ref[...], w1_ref[...], (((1,), (1,)), ((), ())),
                         preferred_element_type=jnp.float32)
    ys_ref[...] = (xw * dinv[:, None]).astype(jnp.bfloat16)
    dinv_ref[...] = dinv


def _tcA(x, W1, degs):
    return pl.pallas_call(
        _tcA_body,
        out_shape=[jax.ShapeDtypeStruct((N, H), jnp.bfloat16),
                   jax.ShapeDtypeStruct((N,), jnp.float32)],
    )(x, W1, degs)


def _tcC_body(parts_ref, dinv_ref, b1_ref, w2_ref, ys2_ref):
    dinv = dinv_ref[...]
    acc = parts_ref[0] + parts_ref[1]
    h1 = jnp.maximum(acc * dinv[:, None] + b1_ref[...][None, :], 0.0)
    xw2 = lax.dot_general(h1, w2_ref[...], (((1,), (1,)), ((), ())),
                          preferred_element_type=jnp.float32)
    ys2_ref[...] = (xw2 * dinv[:, None]).astype(jnp.bfloat16)


def _tcC(parts1, dinv, b1, W2):
    return pl.pallas_call(
        _tcC_body,
        out_shape=jax.ShapeDtypeStruct((N, H), jnp.bfloat16),
    )(parts1, dinv, b1, W2)


def _tcD_body(parts_ref, dinv_ref, b2_ref, batch_ref, emb_ref,
              wah_ref, wae_ref, ba_ref, ws_ref, bs_ref, wo_ref, bo_ref,
              out_ref):
    dinv = dinv_ref[...]
    acc = parts_ref[0] + parts_ref[1]
    h2 = jnp.maximum(acc * dinv[:, None] + b2_ref[...][None, :], 0.0)

    P = lax.dot_general(emb_ref[...], wae_ref[...], (((1,), (1,)), ((), ())),
                        preferred_element_type=jnp.float32)
    batch = batch_ref[...]
    oh = (batch[:, None] == lax.broadcasted_iota(jnp.int32, (1, B), 1)
          ).astype(jnp.float32)
    embp = lax.dot_general(oh, P, (((1,), (0,)), ((), ())),
                           preferred_element_type=jnp.float32)
    ah = lax.dot_general(h2, wah_ref[...], (((1,), (1,)), ((), ())),
                         preferred_element_type=jnp.float32) + embp \
        + ba_ref[...][None, :]
    sc = jnp.where(ah >= 0, ah, 0.2 * ah)
    logits = jnp.sum(sc * ws_ref[...], axis=1, keepdims=True) \
        + bs_ref[...][None, :]
    m = jnp.max(logits)
    e = jnp.exp(logits - m)
    aw = e / jnp.sum(e)
    w = aw * h2
    S = lax.dot_general(oh, w, (((0,), (0,)), ((), ())),
                        preferred_element_type=jnp.float32)
    cnt = jnp.sum(oh, axis=0)
    num = jnp.sum(S * wo_ref[...][0][None, :], axis=1) + cnt * bo_ref[...]
    out_ref[...] = num / jnp.maximum(cnt, 1.0)


def _tcD(parts2, dinv, b2, batch, app_embedding, Wah, Wae, ba, Ws, bs,
         Wo, bo):
    return pl.pallas_call(
        _tcD_body,
        out_shape=jax.ShapeDtypeStruct((B,), jnp.float32),
    )(parts2, dinv, b2, batch, app_embedding, Wah, Wae, ba, Ws, bs,
      Wo, bo)


def kernel(x, edge_index, edge_attr, batch, app_embedding, W1, b1, W2, b2,
           Wa, ba, Ws, bs, Wo, bo):
    src = edge_index[0]
    dst = edge_index[1]
    loop = jnp.arange(N, dtype=src.dtype)
    pad = EPAD - E - N
    zpad = jnp.zeros((pad,), src.dtype)
    srcp = jnp.concatenate([src, loop, zpad])
    dstp = jnp.concatenate([dst, loop, zpad])
    ewp = jnp.concatenate([edge_attr, jnp.ones((N,), edge_attr.dtype),
                           jnp.zeros((pad,), edge_attr.dtype)])
    ewi = lax.bitcast_convert_type(ewp, jnp.int32)
    epk = jnp.stack([srcp, dstp, ewi], 0).reshape(3, NW * NCHUNK, K)
    epk = epk.transpose(1, 0, 2)
    sig = (np.arange(0, H, 32)[:, None]
           + np.stack([np.arange(16), 16 + np.arange(16)], 1).reshape(-1)[None, :]
           ).reshape(-1)
    W1s = W1[sig]
    W2s = W2[sig]
    degs = _deg_kernel(dst, edge_attr)
    ysb1, dinv = _tcA(x, W1s, degs)
    ys1i = lax.bitcast_convert_type(ysb1.reshape(N, H // 2, 2), jnp.int32)
    parts1 = _agg_kernel(epk, ys1i)
    ysb2 = _tcC(parts1, dinv, b1, W2s)
    ys2i = lax.bitcast_convert_type(ysb2.reshape(N, H // 2, 2), jnp.int32)
    parts2 = _agg_kernel(epk, ys2i)
    Wah = Wa[:, :H]
    Wae = Wa[:, H:]
    return _tcD(parts2, dinv, b2, batch, app_embedding, Wah, Wae,
                ba, Ws, bs, Wo, bo)


# speedup vs baseline: 2.1407x; 2.1407x over previous
import functools

import jax
import jax.numpy as jnp
import numpy as np
from jax import lax
from jax.experimental import pallas as pl
from jax.experimental.pallas import tpu as pltpu
from jax.experimental.pallas import tpu_sc as plsc

N = 10000
E = 320000
B = 4
D = 128
H = 128
EMB = 64
ATT = 64

NC = 2
NS = 16
NW = NC * NS
EP_W = E // NW
K = 80
NCHUNK = 133
EPAD = NW * NCHUNK * K
ZROWS = 125
RP_S = N // NS

_mesh = plsc.VectorSubcoreMesh(core_axis_name="c", subcore_axis_name="s")


@functools.partial(
    pl.kernel,
    out_type=jax.ShapeDtypeStruct((NW, N), jnp.float32),
    mesh=_mesh,
    scratch_types=[
        pltpu.VMEM((EP_W,), jnp.int32),
        pltpu.VMEM((EP_W,), jnp.float32),
        pltpu.VMEM((N,), jnp.float32),
    ],
    compiler_params=pltpu.CompilerParams(needs_layout_passes=False, use_tc_tiling_on_sc=False),
)
def _deg_kernel(dst_hbm, ew_hbm, degs_hbm, dst_v, ew_v, deg_v):
    wid = lax.axis_index("s") * NC + lax.axis_index("c")
    base = wid * EP_W

    def zero(i, carry):
        deg_v[pl.ds(i * 16, 16)] = jnp.zeros((16,), jnp.float32)
        return carry

    lax.fori_loop(0, N // 16, zero, 0)

    pltpu.sync_copy(dst_hbm.at[pl.ds(base, EP_W)], dst_v)
    pltpu.sync_copy(ew_hbm.at[pl.ds(base, EP_W)], ew_v)

    def body(j, carry):
        d16 = dst_v[pl.ds(j * 16, 16)]
        w16 = ew_v[pl.ds(j * 16, 16)]
        plsc.addupdate_scatter(deg_v, [d16], w16)
        return carry

    lax.fori_loop(0, EP_W // 16, body, 0)
    pltpu.sync_copy(deg_v, degs_hbm.at[wid])


@functools.partial(
    pl.kernel,
    out_type=jax.ShapeDtypeStruct((NC, N, H), jnp.float32),
    mesh=_mesh,
    scratch_types=(
        [pltpu.VMEM((3, K), jnp.int32)] * 8
        + [pltpu.VMEM((K, H // 2), jnp.int32)] * 4
        + [pltpu.VMEM((K, H), jnp.float32)] * 2
        + [pltpu.VMEM_SHARED((N, H), jnp.float32)]
        + [pltpu.SemaphoreType.DMA] * 14
    ),
    compiler_params=pltpu.CompilerParams(needs_layout_passes=False, use_tc_tiling_on_sc=False),
)
def _agg_kernel(epk_hbm, ys_hbm, parts_hbm, *refs):
    ib = refs[0:8]
    rows = refs[8:12]
    srows = refs[12:14]
    acc_sh = refs[14]
    isem = refs[15:23]
    gsem = refs[23:27]
    ssem = refs[27:29]

    c = lax.axis_index("c")
    s = lax.axis_index("s")
    wid = s * NC + c
    ibase = wid * NCHUNK

    def zero(i, carry):
        for v in range(H // 16):
            srows[0][i, pl.ds(v * 16, 16)] = jnp.zeros((16,), jnp.float32)
        return carry

    lax.fori_loop(0, K, zero, 0)
    for k in range(RP_S // K):
        pltpu.sync_copy(srows[0], acc_sh.at[pl.ds(s * RP_S + k * K, K)])
    rem = RP_S % K
    if rem:
        pltpu.sync_copy(srows[0].at[pl.ds(0, rem)],
                        acc_sh.at[pl.ds(s * RP_S + (RP_S // K) * K, rem)])
    plsc.subcore_barrier()

    def idx_start(cix, p):
        pltpu.async_copy(epk_hbm.at[ibase + cix], ib[p], isem[p])

    def idx_wait(p):
        pltpu.make_async_copy(epk_hbm.at[0], ib[p], isem[p]).wait()

    def gather_start(p, b):
        pltpu.async_copy(ys_hbm.at[ib[p].at[0]], rows[b], gsem[b])

    def gather_wait(p, b):
        pltpu.make_async_copy(ys_hbm.at[ib[p].at[0]], rows[b],
                              gsem[b]).wait()

    def scale(p, b, e):
        two16 = jnp.full((16,), 2, jnp.int32)
        msk = jnp.full((16,), -65536, jnp.int32)

        @plsc.parallel_loop(0, K, unroll=8)
        def body(j):
            idx16 = jnp.zeros((16,), jnp.int32) + j
            raw = plsc.load_gather(ib[p], [two16, idx16])
            bc = plsc.bitcast(raw, jnp.float32)
            for v in range(H // 32):
                pk = rows[b][j, pl.ds(v * 16, 16)]
                lo = plsc.bitcast(pk << 16, jnp.float32)
                hi = plsc.bitcast(pk & msk, jnp.float32)
                srows[e][j, pl.ds(v * 32, 16)] = lo * bc
                srows[e][j, pl.ds(v * 32 + 16, 16)] = hi * bc

    def scatter_start(p, e):
        pltpu.async_copy(srows[e], acc_sh.at[ib[p].at[1]], ssem[e], add=True)

    def scatter_wait(p, e):
        pltpu.make_async_copy(srows[e], acc_sh.at[ib[p].at[1]],
                              ssem[e]).wait()

    def step(cix, m, first=False, last=False, more_idx=True):
        p, b, e = m % 8, m % 4, m % 2
        p2, b2 = (m + 2) % 8, (m + 2) % 4
        if not first:
            scatter_wait((m - 2) % 8, (m - 2) % 2)
        if not last:
            idx_wait(p2)
            gather_start(p2, b2)
        if more_idx:
            idx_start(cix + 5, (m + 5) % 8)
        gather_wait(p, b)
        scale(p, b, e)
        scatter_start(p, e)

    for cix in range(5):
        idx_start(cix, cix)
    idx_wait(0)
    gather_start(0, 0)
    idx_wait(1)
    gather_start(1, 1)
    step(0, 0, first=True)
    step(1, 1, first=True)
    step(2, 2)
    step(3, 3)
    step(4, 4)

    def grp(kk, carry):
        base = 5 + 8 * kk
        for i in range(8):
            step(base + i, 5 + i)
        return carry

    lax.fori_loop(0, (NCHUNK - 5) // 8 - 1, grp, 0)
    for cix in range(NCHUNK - 8, NCHUNK):
        step(cix, cix, last=(cix + 2 >= NCHUNK),
             more_idx=(cix + 5 < NCHUNK))
    scatter_wait((NCHUNK - 2) % 8, (NCHUNK - 2) % 2)
    scatter_wait((NCHUNK - 1) % 8, (NCHUNK - 1) % 2)

    plsc.subcore_barrier()
    for k in range(RP_S // ZROWS):
        off = s * RP_S + k * ZROWS
        pltpu.sync_copy(acc_sh.at[pl.ds(off, ZROWS)],
                        parts_hbm.at[c, pl.ds(off, ZROWS)])


def _tcA_body(x_ref, w1_ref, degs_ref, ys_ref, dinv_ref):
    deg = jnp.sum(degs_ref[...], axis=0) + 1.0
    dinv = lax.rsqrt(deg)
    xw = lax.dot_general(x_ref[...], w1_ref[...], (((1,), (1,)), ((), ())),
                         preferred_element_type=jnp.float32)
    ys_ref[...] = (xw * dinv[:, None]).astype(jnp.bfloat16)
    dinv_ref[...] = dinv


def _tcA(x, W1, degs):
    return pl.pallas_call(
        _tcA_body,
        out_shape=[jax.ShapeDtypeStruct((N, H), jnp.bfloat16),
                   jax.ShapeDtypeStruct((N,), jnp.float32)],
    )(x, W1, degs)


def _tcC_body(parts_ref, dinv_ref, b1_ref, w2_ref, ys2_ref):
    dinv = dinv_ref[...]
    acc = parts_ref[0] + parts_ref[1]
    h1 = jnp.maximum(acc * dinv[:, None] + b1_ref[...][None, :], 0.0)
    xw2 = lax.dot_general(h1, w2_ref[...], (((1,), (1,)), ((), ())),
                          preferred_element_type=jnp.float32)
    ys2_ref[...] = (xw2 * dinv[:, None]).astype(jnp.bfloat16)


def _tcC(parts1, dinv, b1, W2):
    return pl.pallas_call(
        _tcC_body,
        out_shape=jax.ShapeDtypeStruct((N, H), jnp.bfloat16),
    )(parts1, dinv, b1, W2)


def _tcD_body(parts_ref, dinv_ref, b2_ref, batch_ref, emb_ref,
              wah_ref, wae_ref, ba_ref, ws_ref, bs_ref, wo_ref, bo_ref,
              out_ref):
    dinv = dinv_ref[...]
    acc = parts_ref[0] + parts_ref[1]
    h2 = jnp.maximum(acc * dinv[:, None] + b2_ref[...][None, :], 0.0)

    P = lax.dot_general(emb_ref[...], wae_ref[...], (((1,), (1,)), ((), ())),
                        preferred_element_type=jnp.float32)
    batch = batch_ref[...]
    oh = (batch[:, None] == lax.broadcasted_iota(jnp.int32, (1, B), 1)
          ).astype(jnp.float32)
    embp = lax.dot_general(oh, P, (((1,), (0,)), ((), ())),
                           preferred_element_type=jnp.float32)
    ah = lax.dot_general(h2, wah_ref[...], (((1,), (1,)), ((), ())),
                         preferred_element_type=jnp.float32) + embp \
        + ba_ref[...][None, :]
    sc = jnp.where(ah >= 0, ah, 0.2 * ah)
    logits = jnp.sum(sc * ws_ref[...], axis=1, keepdims=True) \
        + bs_ref[...][None, :]
    m = jnp.max(logits)
    e = jnp.exp(logits - m)
    aw = e / jnp.sum(e)
    w = aw * h2
    S = lax.dot_general(oh, w, (((0,), (0,)), ((), ())),
                        preferred_element_type=jnp.float32)
    cnt = jnp.sum(oh, axis=0)
    num = jnp.sum(S * wo_ref[...][0][None, :], axis=1) + cnt * bo_ref[...]
    out_ref[...] = num / jnp.maximum(cnt, 1.0)


def _tcD(parts2, dinv, b2, batch, app_embedding, Wah, Wae, ba, Ws, bs,
         Wo, bo):
    return pl.pallas_call(
        _tcD_body,
        out_shape=jax.ShapeDtypeStruct((B,), jnp.float32),
    )(parts2, dinv, b2, batch, app_embedding, Wah, Wae, ba, Ws, bs,
      Wo, bo)


def kernel(x, edge_index, edge_attr, batch, app_embedding, W1, b1, W2, b2,
           Wa, ba, Ws, bs, Wo, bo):
    src = edge_index[0]
    dst = edge_index[1]
    loop = jnp.arange(N, dtype=src.dtype)
    pad = EPAD - E - N
    zpad = jnp.arange(pad, dtype=src.dtype) % N
    srcp = jnp.concatenate([src, loop, zpad])
    dstp = jnp.concatenate([dst, loop, zpad])
    ewp = jnp.concatenate([edge_attr, jnp.ones((N,), edge_attr.dtype),
                           jnp.zeros((pad,), edge_attr.dtype)])
    ewi = lax.bitcast_convert_type(ewp, jnp.int32)
    epk = jnp.stack([srcp, dstp, ewi], 0).reshape(3, NW * NCHUNK, K)
    epk = epk.transpose(1, 0, 2)
    sig = (np.arange(0, H, 32)[:, None]
           + np.stack([np.arange(16), 16 + np.arange(16)], 1).reshape(-1)[None, :]
           ).reshape(-1)
    W1s = W1[sig]
    W2s = W2[sig]
    degs = _deg_kernel(dst, edge_attr)
    ysb1, dinv = _tcA(x, W1s, degs)
    ys1i = lax.bitcast_convert_type(ysb1.reshape(N, H // 2, 2), jnp.int32)
    parts1 = _agg_kernel(epk, ys1i)
    ysb2 = _tcC(parts1, dinv, b1, W2s)
    ys2i = lax.bitcast_convert_type(ysb2.reshape(N, H // 2, 2), jnp.int32)
    parts2 = _agg_kernel(epk, ys2i)
    Wah = Wa[:, :H]
    Wae = Wa[:, H:]
    return _tcD(parts2, dinv, b2, batch, app_embedding, Wah, Wae,
                ba, Ws, bs, Wo, bo)


# speedup vs baseline: 2.4763x; 1.1568x over previous
import functools

import jax
import jax.numpy as jnp
import numpy as np
from jax import lax
from jax.experimental import pallas as pl
from jax.experimental.pallas import tpu as pltpu
from jax.experimental.pallas import tpu_sc as plsc

N = 10000
E = 320000
B = 4
D = 128
H = 128
EMB = 64
ATT = 64

NC = 2
NS = 16
NW = NC * NS
EP_W = E // NW
K = 80
NCHUNK = 133
EPAD = NW * NCHUNK * K
ZROWS = 125
RP_S = N // NS

_mesh = plsc.VectorSubcoreMesh(core_axis_name="c", subcore_axis_name="s")


@functools.partial(
    pl.kernel,
    out_type=jax.ShapeDtypeStruct((NW, N), jnp.float32),
    mesh=_mesh,
    scratch_types=[
        pltpu.VMEM((EP_W,), jnp.int32),
        pltpu.VMEM((EP_W,), jnp.float32),
        pltpu.VMEM((N,), jnp.float32),
    ],
    compiler_params=pltpu.CompilerParams(needs_layout_passes=False, use_tc_tiling_on_sc=False),
)
def _deg_kernel(dst_hbm, ew_hbm, degs_hbm, dst_v, ew_v, deg_v):
    wid = lax.axis_index("s") * NC + lax.axis_index("c")
    base = wid * EP_W

    def zero(i, carry):
        deg_v[pl.ds(i * 16, 16)] = jnp.zeros((16,), jnp.float32)
        return carry

    lax.fori_loop(0, N // 16, zero, 0)

    pltpu.sync_copy(dst_hbm.at[pl.ds(base, EP_W)], dst_v)
    pltpu.sync_copy(ew_hbm.at[pl.ds(base, EP_W)], ew_v)

    def body(j, carry):
        d16 = dst_v[pl.ds(j * 16, 16)]
        w16 = ew_v[pl.ds(j * 16, 16)]
        plsc.addupdate_scatter(deg_v, [d16], w16)
        return carry

    lax.fori_loop(0, EP_W // 16, body, 0)
    pltpu.sync_copy(deg_v, degs_hbm.at[wid])


@functools.partial(
    pl.kernel,
    out_type=jax.ShapeDtypeStruct((NC, N, H), jnp.float32),
    mesh=_mesh,
    scratch_types=(
        [pltpu.VMEM((3, K), jnp.int32)] * 8
        + [pltpu.VMEM((K, H // 2), jnp.int32)] * 4
        + [pltpu.VMEM((K, H), jnp.float32)] * 2
        + [pltpu.VMEM_SHARED((N, H), jnp.float32)]
        + [pltpu.SemaphoreType.DMA] * 14
    ),
    compiler_params=pltpu.CompilerParams(needs_layout_passes=False, use_tc_tiling_on_sc=False),
)
def _agg_kernel(epk_hbm, ys_hbm, parts_hbm, *refs):
    ib = refs[0:8]
    rows = refs[8:12]
    srows = refs[12:14]
    acc_sh = refs[14]
    isem = refs[15:23]
    gsem = refs[23:27]
    ssem = refs[27:29]

    c = lax.axis_index("c")
    s = lax.axis_index("s")
    wid = s * NC + c
    ibase = wid * NCHUNK

    def zero(i, carry):
        for v in range(H // 16):
            srows[0][i, pl.ds(v * 16, 16)] = jnp.zeros((16,), jnp.float32)
        return carry

    lax.fori_loop(0, K, zero, 0)
    for k in range(RP_S // K):
        pltpu.sync_copy(srows[0], acc_sh.at[pl.ds(s * RP_S + k * K, K)])
    rem = RP_S % K
    if rem:
        pltpu.sync_copy(srows[0].at[pl.ds(0, rem)],
                        acc_sh.at[pl.ds(s * RP_S + (RP_S // K) * K, rem)])
    plsc.subcore_barrier()

    def idx_start(cix, p):
        pltpu.async_copy(epk_hbm.at[ibase + cix], ib[p], isem[p])

    def idx_wait(p):
        pltpu.make_async_copy(epk_hbm.at[0], ib[p], isem[p]).wait()

    def gather_start(p, b):
        pltpu.async_copy(ys_hbm.at[ib[p].at[0]], rows[b], gsem[b])

    def gather_wait(p, b):
        pltpu.make_async_copy(ys_hbm.at[ib[p].at[0]], rows[b],
                              gsem[b]).wait()

    def scale(p, b, e):
        two16 = jnp.full((16,), 2, jnp.int32)
        msk = jnp.full((16,), -65536, jnp.int32)

        @plsc.parallel_loop(0, K, unroll=8)
        def body(j):
            idx16 = jnp.zeros((16,), jnp.int32) + j
            raw = plsc.load_gather(ib[p], [two16, idx16])
            bc = plsc.bitcast(raw, jnp.float32)
            for v in range(H // 32):
                pk = rows[b][j, pl.ds(v * 16, 16)]
                lo = plsc.bitcast(pk << 16, jnp.float32)
                hi = plsc.bitcast(pk & msk, jnp.float32)
                srows[e][j, pl.ds(v * 32, 16)] = lo * bc
                srows[e][j, pl.ds(v * 32 + 16, 16)] = hi * bc

    def scatter_start(p, e):
        pltpu.async_copy(srows[e], acc_sh.at[ib[p].at[1]], ssem[e], add=True)

    def scatter_wait(p, e):
        pltpu.make_async_copy(srows[e], acc_sh.at[ib[p].at[1]],
                              ssem[e]).wait()

    def step(cix, m, first=False, last=False, more_idx=True):
        p, b, e = m % 8, m % 4, m % 2
        p2, b2 = (m + 2) % 8, (m + 2) % 4
        if not first:
            scatter_wait((m - 2) % 8, (m - 2) % 2)
        if not last:
            idx_wait(p2)
            gather_start(p2, b2)
        if more_idx:
            idx_start(cix + 5, (m + 5) % 8)
        gather_wait(p, b)
        scale(p, b, e)
        scatter_start(p, e)

    for cix in range(5):
        idx_start(cix, cix)
    idx_wait(0)
    gather_start(0, 0)
    idx_wait(1)
    gather_start(1, 1)
    step(0, 0, first=True)
    step(1, 1, first=True)
    step(2, 2)
    step(3, 3)
    step(4, 4)

    def grp(kk, carry):
        base = 5 + 8 * kk
        for i in range(8):
            step(base + i, 5 + i)
        return carry

    lax.fori_loop(0, (NCHUNK - 5) // 8 - 1, grp, 0)
    for cix in range(NCHUNK - 8, NCHUNK):
        step(cix, cix, last=(cix + 2 >= NCHUNK),
             more_idx=(cix + 5 < NCHUNK))
    scatter_wait((NCHUNK - 2) % 8, (NCHUNK - 2) % 2)
    scatter_wait((NCHUNK - 1) % 8, (NCHUNK - 1) % 2)

    plsc.subcore_barrier()
    for k in range(RP_S // ZROWS):
        off = s * RP_S + k * ZROWS
        pltpu.sync_copy(acc_sh.at[pl.ds(off, ZROWS)],
                        parts_hbm.at[c, pl.ds(off, ZROWS)])


def _pack_bf16_pairs(ys):
    bits_lo = lax.bitcast_convert_type(ys[:, :H // 2], jnp.int32)
    bits_hi = lax.bitcast_convert_type(ys[:, H // 2:], jnp.int32)
    lo = ((bits_lo + 0x8000) >> 16) & 0xFFFF
    hi = (bits_hi + 0x8000) & -65536
    return lo | hi


def _tcA_body(x_ref, w1_ref, degs_ref, ys_ref, dinv_ref):
    deg = jnp.sum(degs_ref[...], axis=0) + 1.0
    dinv = lax.rsqrt(deg)
    xw = lax.dot_general(x_ref[...], w1_ref[...], (((1,), (1,)), ((), ())),
                         preferred_element_type=jnp.float32)
    ys_ref[...] = _pack_bf16_pairs(xw * dinv[:, None])
    dinv_ref[...] = dinv


def _tcA(x, W1, degs):
    return pl.pallas_call(
        _tcA_body,
        out_shape=[jax.ShapeDtypeStruct((N, H // 2), jnp.int32),
                   jax.ShapeDtypeStruct((N,), jnp.float32)],
    )(x, W1, degs)


def _tcC_body(parts_ref, dinv_ref, b1_ref, w2_ref, ys2_ref):
    dinv = dinv_ref[...]
    acc = parts_ref[0] + parts_ref[1]
    h1 = jnp.maximum(acc * dinv[:, None] + b1_ref[...][None, :], 0.0)
    xw2 = lax.dot_general(h1, w2_ref[...], (((1,), (1,)), ((), ())),
                          preferred_element_type=jnp.float32)
    ys2_ref[...] = _pack_bf16_pairs(xw2 * dinv[:, None])


def _tcC(parts1, dinv, b1, W2):
    return pl.pallas_call(
        _tcC_body,
        out_shape=jax.ShapeDtypeStruct((N, H // 2), jnp.int32),
    )(parts1, dinv, b1, W2)


def _tcD_body(parts_ref, dinv_ref, b2_ref, batch_ref, emb_ref,
              wah_ref, wae_ref, ba_ref, ws_ref, bs_ref, wo_ref, bo_ref,
              out_ref):
    dinv = dinv_ref[...]
    acc = parts_ref[0] + parts_ref[1]
    h2 = jnp.maximum(acc * dinv[:, None] + b2_ref[...][None, :], 0.0)

    P = lax.dot_general(emb_ref[...], wae_ref[...], (((1,), (1,)), ((), ())),
                        preferred_element_type=jnp.float32)
    batch = batch_ref[...]
    oh = (batch[:, None] == lax.broadcasted_iota(jnp.int32, (1, B), 1)
          ).astype(jnp.float32)
    embp = lax.dot_general(oh, P, (((1,), (0,)), ((), ())),
                           preferred_element_type=jnp.float32)
    ah = lax.dot_general(h2, wah_ref[...], (((1,), (1,)), ((), ())),
                         preferred_element_type=jnp.float32) + embp \
        + ba_ref[...][None, :]
    sc = jnp.where(ah >= 0, ah, 0.2 * ah)
    logits = jnp.sum(sc * ws_ref[...], axis=1, keepdims=True) \
        + bs_ref[...][None, :]
    m = jnp.max(logits)
    e = jnp.exp(logits - m)
    aw = e / jnp.sum(e)
    w = aw * h2
    S = lax.dot_general(oh, w, (((0,), (0,)), ((), ())),
                        preferred_element_type=jnp.float32)
    cnt = jnp.sum(oh, axis=0)
    num = jnp.sum(S * wo_ref[...][0][None, :], axis=1) + cnt * bo_ref[...]
    out_ref[...] = num / jnp.maximum(cnt, 1.0)


def _tcD(parts2, dinv, b2, batch, app_embedding, Wah, Wae, ba, Ws, bs,
         Wo, bo):
    return pl.pallas_call(
        _tcD_body,
        out_shape=jax.ShapeDtypeStruct((B,), jnp.float32),
    )(parts2, dinv, b2, batch, app_embedding, Wah, Wae, ba, Ws, bs,
      Wo, bo)


def kernel(x, edge_index, edge_attr, batch, app_embedding, W1, b1, W2, b2,
           Wa, ba, Ws, bs, Wo, bo):
    src = edge_index[0]
    dst = edge_index[1]
    loop = jnp.arange(N, dtype=src.dtype)
    pad = EPAD - E - N
    zpad = jnp.arange(pad, dtype=src.dtype) % N
    srcp = jnp.concatenate([src, loop, zpad])
    dstp = jnp.concatenate([dst, loop, zpad])
    ewp = jnp.concatenate([edge_attr, jnp.ones((N,), edge_attr.dtype),
                           jnp.zeros((pad,), edge_attr.dtype)])
    ewi = lax.bitcast_convert_type(ewp, jnp.int32)
    epk = jnp.stack([srcp, dstp, ewi], 0).reshape(3, NW * NCHUNK, K)
    epk = epk.transpose(1, 0, 2)
    ordr = np.concatenate([np.arange(16) + 32 * v for v in range(4)]
                          + [np.arange(16) + 32 * v + 16 for v in range(4)])
    W1s = W1[ordr]
    W2s = W2[ordr]
    degs = _deg_kernel(dst, edge_attr)
    ys1i, dinv = _tcA(x, W1s, degs)
    parts1 = _agg_kernel(epk, ys1i)
    ys2i = _tcC(parts1, dinv, b1, W2s)
    parts2 = _agg_kernel(epk, ys2i)
    Wah = Wa[:, :H]
    Wae = Wa[:, H:]
    return _tcD(parts2, dinv, b2, batch, app_embedding, Wah, Wae,
                ba, Ws, bs, Wo, bo)


# speedup vs baseline: 2.8313x; 1.1433x over previous
import functools

import jax
import jax.numpy as jnp
import numpy as np
from jax import lax
from jax.experimental import pallas as pl
from jax.experimental.pallas import tpu as pltpu
from jax.experimental.pallas import tpu_sc as plsc

N = 10000
E = 320000
B = 4
D = 128
H = 128
EMB = 64
ATT = 64

NC = 2
NS = 16
NW = NC * NS
EP_W = E // NW
K = 80
NCHUNK = 133
EPAD = NW * NCHUNK * K
ZROWS = 125
RP_S = N // NS

_mesh = plsc.VectorSubcoreMesh(core_axis_name="c", subcore_axis_name="s")


NCH0 = EP_W // K
TAILC = NCHUNK - NCH0


def _build_tail():
    ids = np.concatenate([np.arange(N, dtype=np.int32),
                          np.arange(NW * TAILC * K - N,
                                    dtype=np.int32) % N])
    ew = np.concatenate([np.full((N,), 0x3F800000, np.int32),
                         np.zeros((NW * TAILC * K - N,), np.int32)])
    t = np.stack([ids, ids, ew], 0).reshape(3, NW * TAILC, K)
    return np.ascontiguousarray(t.transpose(1, 0, 2))


_TAIL = _build_tail()


@functools.partial(
    pl.kernel,
    out_type=[jax.ShapeDtypeStruct((NW, N), jnp.float32),
              jax.ShapeDtypeStruct((NW * NCHUNK, 3, K), jnp.int32)],
    mesh=_mesh,
    scratch_types=[
        pltpu.VMEM((NCH0, K), jnp.int32),
        pltpu.VMEM((NCH0, K), jnp.int32),
        pltpu.VMEM((NCH0, K), jnp.int32),
        pltpu.VMEM((N,), jnp.float32),
    ],
    compiler_params=pltpu.CompilerParams(needs_layout_passes=False, use_tc_tiling_on_sc=False),
)
def _deg_kernel(src_hbm, dst_hbm, ewi_hbm, tail_hbm, degs_hbm, epk_hbm,
                src_v, dst_v, ewi_v, deg_v):
    wid = lax.axis_index("s") * NC + lax.axis_index("c")

    def zero(i, carry):
        deg_v[pl.ds(i * 16, 16)] = jnp.zeros((16,), jnp.float32)
        return carry

    lax.fori_loop(0, N // 16, zero, 0)

    pltpu.sync_copy(src_hbm.at[wid], src_v)
    pltpu.sync_copy(dst_hbm.at[wid], dst_v)
    pltpu.sync_copy(ewi_hbm.at[wid], ewi_v)

    def body(i, carry):
        for jj in range(K // 16):
            d16 = dst_v[i, pl.ds(jj * 16, 16)]
            w16 = plsc.bitcast(ewi_v[i, pl.ds(jj * 16, 16)], jnp.float32)
            plsc.addupdate_scatter(deg_v, [d16], w16)
        return carry

    lax.fori_loop(0, NCH0, body, 0)
    pltpu.sync_copy(deg_v, degs_hbm.at[wid])

    cbase = wid * NCHUNK
    pltpu.sync_copy(src_v, epk_hbm.at[pl.ds(cbase, NCH0), 0])
    pltpu.sync_copy(dst_v, epk_hbm.at[pl.ds(cbase, NCH0), 1])
    pltpu.sync_copy(ewi_v, epk_hbm.at[pl.ds(cbase, NCH0), 2])
    pltpu.sync_copy(tail_hbm.at[pl.ds(wid * TAILC, TAILC)],
                    epk_hbm.at[pl.ds(cbase + NCH0, TAILC)])


@functools.partial(
    pl.kernel,
    out_type=jax.ShapeDtypeStruct((NC, N, H), jnp.float32),
    mesh=_mesh,
    scratch_types=(
        [pltpu.VMEM((3, K), jnp.int32)] * 8
        + [pltpu.VMEM((K, H // 2), jnp.int32)] * 4
        + [pltpu.VMEM((K, H), jnp.float32)] * 2
        + [pltpu.VMEM_SHARED((N, H), jnp.float32)]
        + [pltpu.SemaphoreType.DMA] * 14
    ),
    compiler_params=pltpu.CompilerParams(needs_layout_passes=False, use_tc_tiling_on_sc=False),
)
def _agg_kernel(epk_hbm, ys_hbm, parts_hbm, *refs):
    ib = refs[0:8]
    rows = refs[8:12]
    srows = refs[12:14]
    acc_sh = refs[14]
    isem = refs[15:23]
    gsem = refs[23:27]
    ssem = refs[27:29]

    c = lax.axis_index("c")
    s = lax.axis_index("s")
    wid = s * NC + c
    ibase = wid * NCHUNK

    def zero(i, carry):
        for v in range(H // 16):
            srows[0][i, pl.ds(v * 16, 16)] = jnp.zeros((16,), jnp.float32)
        return carry

    lax.fori_loop(0, K, zero, 0)
    for k in range(RP_S // K):
        pltpu.sync_copy(srows[0], acc_sh.at[pl.ds(s * RP_S + k * K, K)])
    rem = RP_S % K
    if rem:
        pltpu.sync_copy(srows[0].at[pl.ds(0, rem)],
                        acc_sh.at[pl.ds(s * RP_S + (RP_S // K) * K, rem)])
    plsc.subcore_barrier()

    def idx_start(cix, p):
        pltpu.async_copy(epk_hbm.at[ibase + cix], ib[p], isem[p])

    def idx_wait(p):
        pltpu.make_async_copy(epk_hbm.at[0], ib[p], isem[p]).wait()

    def gather_start(p, b):
        pltpu.async_copy(ys_hbm.at[ib[p].at[0]], rows[b], gsem[b])

    def gather_wait(p, b):
        pltpu.make_async_copy(ys_hbm.at[ib[p].at[0]], rows[b],
                              gsem[b]).wait()

    def scale(p, b, e):
        two16 = jnp.full((16,), 2, jnp.int32)
        msk = jnp.full((16,), -65536, jnp.int32)

        @plsc.parallel_loop(0, K, unroll=8)
        def body(j):
            idx16 = jnp.zeros((16,), jnp.int32) + j
            raw = plsc.load_gather(ib[p], [two16, idx16])
            bc = plsc.bitcast(raw, jnp.float32)
            for v in range(H // 32):
                pk = rows[b][j, pl.ds(v * 16, 16)]
                lo = plsc.bitcast(pk << 16, jnp.float32)
                hi = plsc.bitcast(pk & msk, jnp.float32)
                srows[e][j, pl.ds(v * 32, 16)] = lo * bc
                srows[e][j, pl.ds(v * 32 + 16, 16)] = hi * bc

    def scatter_start(p, e):
        pltpu.async_copy(srows[e], acc_sh.at[ib[p].at[1]], ssem[e], add=True)

    def scatter_wait(p, e):
        pltpu.make_async_copy(srows[e], acc_sh.at[ib[p].at[1]],
                              ssem[e]).wait()

    def step(cix, m, first=False, last=False, more_idx=True):
        p, b, e = m % 8, m % 4, m % 2
        p2, b2 = (m + 2) % 8, (m + 2) % 4
        if not first:
            scatter_wait((m - 2) % 8, (m - 2) % 2)
        if not last:
            idx_wait(p2)
            gather_start(p2, b2)
        if more_idx:
            idx_start(cix + 5, (m + 5) % 8)
        gather_wait(p, b)
        scale(p, b, e)
        scatter_start(p, e)

    for cix in range(5):
        idx_start(cix, cix)
    idx_wait(0)
    gather_start(0, 0)
    idx_wait(1)
    gather_start(1, 1)
    step(0, 0, first=True)
    step(1, 1, first=True)
    step(2, 2)
    step(3, 3)
    step(4, 4)

    def grp(kk, carry):
        base = 5 + 8 * kk
        for i in range(8):
            step(base + i, 5 + i)
        return carry

    lax.fori_loop(0, (NCHUNK - 5) // 8 - 1, grp, 0)
    for cix in range(NCHUNK - 8, NCHUNK):
        step(cix, cix, last=(cix + 2 >= NCHUNK),
             more_idx=(cix + 5 < NCHUNK))
    scatter_wait((NCHUNK - 2) % 8, (NCHUNK - 2) % 2)
    scatter_wait((NCHUNK - 1) % 8, (NCHUNK - 1) % 2)

    plsc.subcore_barrier()
    for k in range(RP_S // ZROWS):
        off = s * RP_S + k * ZROWS
        pltpu.sync_copy(acc_sh.at[pl.ds(off, ZROWS)],
                        parts_hbm.at[c, pl.ds(off, ZROWS)])


def _pack_bf16_pairs(ys):
    bits_lo = lax.bitcast_convert_type(ys[:, :H // 2], jnp.int32)
    bits_hi = lax.bitcast_convert_type(ys[:, H // 2:], jnp.int32)
    lo = ((bits_lo + 0x8000) >> 16) & 0xFFFF
    hi = (bits_hi + 0x8000) & -65536
    return lo | hi


def _tcA_body(x_ref, w1_ref, degs_ref, ys_ref, dinv_ref):
    deg = jnp.sum(degs_ref[...], axis=0) + 1.0
    dinv = lax.rsqrt(deg)
    xw = lax.dot_general(x_ref[...], w1_ref[...], (((1,), (1,)), ((), ())),
                         preferred_element_type=jnp.float32)
    ys_ref[...] = _pack_bf16_pairs(xw * dinv[:, None])
    dinv_ref[...] = dinv


def _tcA(x, W1, degs):
    return pl.pallas_call(
        _tcA_body,
        out_shape=[jax.ShapeDtypeStruct((N, H // 2), jnp.int32),
                   jax.ShapeDtypeStruct((N,), jnp.float32)],
    )(x, W1, degs)


def _tcC_body(parts_ref, dinv_ref, b1_ref, w2_ref, ys2_ref):
    dinv = dinv_ref[...]
    acc = parts_ref[0] + parts_ref[1]
    h1 = jnp.maximum(acc * dinv[:, None] + b1_ref[...][None, :], 0.0)
    xw2 = lax.dot_general(h1, w2_ref[...], (((1,), (1,)), ((), ())),
                          preferred_element_type=jnp.float32)
    ys2_ref[...] = _pack_bf16_pairs(xw2 * dinv[:, None])


def _tcC(parts1, dinv, b1, W2):
    return pl.pallas_call(
        _tcC_body,
        out_shape=jax.ShapeDtypeStruct((N, H // 2), jnp.int32),
    )(parts1, dinv, b1, W2)


def _tcD_body(parts_ref, dinv_ref, b2_ref, batch_ref, emb_ref,
              wah_ref, wae_ref, ba_ref, ws_ref, bs_ref, wo_ref, bo_ref,
              out_ref):
    dinv = dinv_ref[...]
    acc = parts_ref[0] + parts_ref[1]
    h2 = jnp.maximum(acc * dinv[:, None] + b2_ref[...][None, :], 0.0)

    P = lax.dot_general(emb_ref[...], wae_ref[...], (((1,), (1,)), ((), ())),
                        preferred_element_type=jnp.float32)
    batch = batch_ref[...]
    oh = (batch[:, None] == lax.broadcasted_iota(jnp.int32, (1, B), 1)
          ).astype(jnp.float32)
    embp = lax.dot_general(oh, P, (((1,), (0,)), ((), ())),
                           preferred_element_type=jnp.float32)
    ah = lax.dot_general(h2, wah_ref[...], (((1,), (1,)), ((), ())),
                         preferred_element_type=jnp.float32) + embp \
        + ba_ref[...][None, :]
    sc = jnp.where(ah >= 0, ah, 0.2 * ah)
    logits = jnp.sum(sc * ws_ref[...], axis=1, keepdims=True) \
        + bs_ref[...][None, :]
    m = jnp.max(logits)
    e = jnp.exp(logits - m)
    aw = e / jnp.sum(e)
    w = aw * h2
    S = lax.dot_general(oh, w, (((0,), (0,)), ((), ())),
                        preferred_element_type=jnp.float32)
    cnt = jnp.sum(oh, axis=0)
    num = jnp.sum(S * wo_ref[...][0][None, :], axis=1) + cnt * bo_ref[...]
    out_ref[...] = num / jnp.maximum(cnt, 1.0)


def _tcD(parts2, dinv, b2, batch, app_embedding, Wah, Wae, ba, Ws, bs,
         Wo, bo):
    return pl.pallas_call(
        _tcD_body,
        out_shape=jax.ShapeDtypeStruct((B,), jnp.float32),
    )(parts2, dinv, b2, batch, app_embedding, Wah, Wae, ba, Ws, bs,
      Wo, bo)


def kernel(x, edge_index, edge_attr, batch, app_embedding, W1, b1, W2, b2,
           Wa, ba, Ws, bs, Wo, bo):
    src = edge_index[0]
    dst = edge_index[1]
    ewi = lax.bitcast_convert_type(edge_attr, jnp.int32)
    ordr = np.concatenate([np.arange(16) + 32 * v for v in range(4)]
                          + [np.arange(16) + 32 * v + 16 for v in range(4)])
    W1s = W1[ordr]
    W2s = W2[ordr]
    degs, epk = _deg_kernel(src.reshape(NW, NCH0, K),
                            dst.reshape(NW, NCH0, K),
                            ewi.reshape(NW, NCH0, K),
                            jnp.asarray(_TAIL))
    ys1i, dinv = _tcA(x, W1s, degs)
    parts1 = _agg_kernel(epk, ys1i)
    ys2i = _tcC(parts1, dinv, b1, W2s)
    parts2 = _agg_kernel(epk, ys2i)
    Wah = Wa[:, :H]
    Wae = Wa[:, H:]
    return _tcD(parts2, dinv, b2, batch, app_embedding, Wah, Wae,
                ba, Ws, bs, Wo, bo)


# speedup vs baseline: 2.9121x; 1.0285x over previous
import functools

import jax
import jax.numpy as jnp
import numpy as np
from jax import lax
from jax.experimental import pallas as pl
from jax.experimental.pallas import tpu as pltpu
from jax.experimental.pallas import tpu_sc as plsc

N = 10000
E = 320000
B = 4
D = 128
H = 128
EMB = 64
ATT = 64

NC = 2
NS = 16
NW = NC * NS
EP_W = E // NW
K = 80
NCHUNK = 133
EPAD = NW * NCHUNK * K
ZROWS = 125
RP_S = N // NS

_mesh = plsc.VectorSubcoreMesh(core_axis_name="c", subcore_axis_name="s")


NCH0 = EP_W // K
TAILC = NCHUNK - NCH0


def _build_tail():
    ids = np.concatenate([np.arange(N, dtype=np.int32),
                          np.arange(NW * TAILC * K - N,
                                    dtype=np.int32) % N])
    ew = np.concatenate([np.full((N,), 0x3F800000, np.int32),
                         np.zeros((NW * TAILC * K - N,), np.int32)])
    t = np.stack([ids, ids, ew], 0).reshape(3, NW * TAILC, K)
    return np.ascontiguousarray(t.transpose(1, 0, 2))


_TAIL = _build_tail()


@functools.partial(
    pl.kernel,
    out_type=[jax.ShapeDtypeStruct((NW, N), jnp.float32),
              jax.ShapeDtypeStruct((NW * NCHUNK, 3, K), jnp.int32)],
    mesh=_mesh,
    scratch_types=[
        pltpu.VMEM((NCH0, K), jnp.int32),
        pltpu.VMEM((NCH0, K), jnp.int32),
        pltpu.VMEM((NCH0, K), jnp.int32),
        pltpu.VMEM((N,), jnp.float32),
    ],
    compiler_params=pltpu.CompilerParams(needs_layout_passes=False, use_tc_tiling_on_sc=False),
)
def _deg_kernel(src_hbm, dst_hbm, ewi_hbm, tail_hbm, degs_hbm, epk_hbm,
                src_v, dst_v, ewi_v, deg_v):
    wid = lax.axis_index("s") * NC + lax.axis_index("c")

    def zero(i, carry):
        deg_v[pl.ds(i * 16, 16)] = jnp.zeros((16,), jnp.float32)
        return carry

    lax.fori_loop(0, N // 16, zero, 0)

    pltpu.sync_copy(src_hbm.at[wid], src_v)
    pltpu.sync_copy(dst_hbm.at[wid], dst_v)
    pltpu.sync_copy(ewi_hbm.at[wid], ewi_v)

    def body(i, carry):
        for jj in range(K // 16):
            d16 = dst_v[i, pl.ds(jj * 16, 16)]
            w16 = plsc.bitcast(ewi_v[i, pl.ds(jj * 16, 16)], jnp.float32)
            plsc.addupdate_scatter(deg_v, [d16], w16)
        return carry

    lax.fori_loop(0, NCH0, body, 0)
    pltpu.sync_copy(deg_v, degs_hbm.at[wid])

    cbase = wid * NCHUNK
    pltpu.sync_copy(src_v, epk_hbm.at[pl.ds(cbase, NCH0), 0])
    pltpu.sync_copy(dst_v, epk_hbm.at[pl.ds(cbase, NCH0), 1])
    pltpu.sync_copy(ewi_v, epk_hbm.at[pl.ds(cbase, NCH0), 2])
    pltpu.sync_copy(tail_hbm.at[pl.ds(wid * TAILC, TAILC)],
                    epk_hbm.at[pl.ds(cbase + NCH0, TAILC)])


@functools.partial(
    pl.kernel,
    out_type=jax.ShapeDtypeStruct((NC, N, H), jnp.float32),
    mesh=_mesh,
    scratch_types=(
        [pltpu.VMEM((3, K), jnp.int32)] * 8
        + [pltpu.VMEM((K, H // 2), jnp.int32)] * 4
        + [pltpu.VMEM((K, H), jnp.float32)] * 2
        + [pltpu.VMEM_SHARED((N, H), jnp.float32)]
        + [pltpu.SemaphoreType.DMA] * 14
    ),
    compiler_params=pltpu.CompilerParams(needs_layout_passes=False, use_tc_tiling_on_sc=False),
)
def _agg_kernel(epk_hbm, ys_hbm, parts_hbm, *refs):
    ib = refs[0:8]
    rows = refs[8:12]
    srows = refs[12:14]
    acc_sh = refs[14]
    isem = refs[15:23]
    gsem = refs[23:27]
    ssem = refs[27:29]

    c = lax.axis_index("c")
    s = lax.axis_index("s")
    wid = s * NC + c
    ibase = wid * NCHUNK

    def zero(i, carry):
        for v in range(H // 16):
            srows[0][i, pl.ds(v * 16, 16)] = jnp.zeros((16,), jnp.float32)
        return carry

    lax.fori_loop(0, K, zero, 0)
    for k in range(RP_S // K):
        pltpu.sync_copy(srows[0], acc_sh.at[pl.ds(s * RP_S + k * K, K)])
    rem = RP_S % K
    if rem:
        pltpu.sync_copy(srows[0].at[pl.ds(0, rem)],
                        acc_sh.at[pl.ds(s * RP_S + (RP_S // K) * K, rem)])
    plsc.subcore_barrier()

    def idx_start(cix, p):
        pltpu.async_copy(epk_hbm.at[ibase + cix], ib[p], isem[p])

    def idx_wait(p):
        pltpu.make_async_copy(epk_hbm.at[0], ib[p], isem[p]).wait()

    def gather_start(p, b):
        pltpu.async_copy(ys_hbm.at[ib[p].at[0]], rows[b], gsem[b])

    def gather_wait(p, b):
        pltpu.make_async_copy(ys_hbm.at[ib[p].at[0]], rows[b],
                              gsem[b]).wait()

    def scale(p, b, e):
        two16 = jnp.full((16,), 2, jnp.int32)
        msk = jnp.full((16,), -65536, jnp.int32)

        @plsc.parallel_loop(0, K, unroll=16)
        def body(j):
            idx16 = jnp.zeros((16,), jnp.int32) + j
            raw = plsc.load_gather(ib[p], [two16, idx16])
            bc = plsc.bitcast(raw, jnp.float32)
            for v in range(H // 32):
                pk = rows[b][j, pl.ds(v * 16, 16)]
                lo = plsc.bitcast(pk << 16, jnp.float32)
                hi = plsc.bitcast(pk & msk, jnp.float32)
                srows[e][j, pl.ds(v * 32, 16)] = lo * bc
                srows[e][j, pl.ds(v * 32 + 16, 16)] = hi * bc

    def scatter_start(p, e):
        pltpu.async_copy(srows[e], acc_sh.at[ib[p].at[1]], ssem[e], add=True)

    def scatter_wait(p, e):
        pltpu.make_async_copy(srows[e], acc_sh.at[ib[p].at[1]],
                              ssem[e]).wait()

    def step(cix, m, first=False, last=False, more_idx=True):
        p, b, e = m % 8, m % 4, m % 2
        p2, b2 = (m + 2) % 8, (m + 2) % 4
        if not first:
            scatter_wait((m - 2) % 8, (m - 2) % 2)
        if not last:
            idx_wait(p2)
            gather_start(p2, b2)
        if more_idx:
            idx_start(cix + 5, (m + 5) % 8)
        gather_wait(p, b)
        scale(p, b, e)
        scatter_start(p, e)

    for cix in range(5):
        idx_start(cix, cix)
    idx_wait(0)
    gather_start(0, 0)
    idx_wait(1)
    gather_start(1, 1)
    step(0, 0, first=True)
    step(1, 1, first=True)
    step(2, 2)
    step(3, 3)
    step(4, 4)

    def grp(kk, carry):
        base = 5 + 8 * kk
        for i in range(8):
            step(base + i, 5 + i)
        return carry

    lax.fori_loop(0, (NCHUNK - 5) // 8 - 1, grp, 0)
    for cix in range(NCHUNK - 8, NCHUNK):
        step(cix, cix, last=(cix + 2 >= NCHUNK),
             more_idx=(cix + 5 < NCHUNK))
    scatter_wait((NCHUNK - 2) % 8, (NCHUNK - 2) % 2)
    scatter_wait((NCHUNK - 1) % 8, (NCHUNK - 1) % 2)

    plsc.subcore_barrier()
    for k in range(RP_S // ZROWS):
        off = s * RP_S + k * ZROWS
        pltpu.sync_copy(acc_sh.at[pl.ds(off, ZROWS)],
                        parts_hbm.at[c, pl.ds(off, ZROWS)])


def _pack_bf16_pairs(ys):
    bits_lo = lax.bitcast_convert_type(ys[:, :H // 2], jnp.int32)
    bits_hi = lax.bitcast_convert_type(ys[:, H // 2:], jnp.int32)
    lo = ((bits_lo + 0x8000) >> 16) & 0xFFFF
    hi = (bits_hi + 0x8000) & -65536
    return lo | hi


def _tcA_body(x_ref, w1_ref, degs_ref, ys_ref, dinv_ref):
    deg = jnp.sum(degs_ref[...], axis=0) + 1.0
    dinv = lax.rsqrt(deg)
    xw = lax.dot_general(x_ref[...], w1_ref[...], (((1,), (1,)), ((), ())),
                         preferred_element_type=jnp.float32)
    ys_ref[...] = _pack_bf16_pairs(xw * dinv[:, None])
    dinv_ref[...] = dinv


def _tcA(x, W1, degs):
    return pl.pallas_call(
        _tcA_body,
        out_shape=[jax.ShapeDtypeStruct((N, H // 2), jnp.int32),
                   jax.ShapeDtypeStruct((N,), jnp.float32)],
    )(x, W1, degs)


def _tcC_body(parts_ref, dinv_ref, b1_ref, w2_ref, ys2_ref):
    dinv = dinv_ref[...]
    acc = parts_ref[0] + parts_ref[1]
    h1 = jnp.maximum(acc * dinv[:, None] + b1_ref[...][None, :], 0.0)
    xw2 = lax.dot_general(h1, w2_ref[...], (((1,), (1,)), ((), ())),
                          preferred_element_type=jnp.float32)
    ys2_ref[...] = _pack_bf16_pairs(xw2 * dinv[:, None])


def _tcC(parts1, dinv, b1, W2):
    return pl.pallas_call(
        _tcC_body,
        out_shape=jax.ShapeDtypeStruct((N, H // 2), jnp.int32),
    )(parts1, dinv, b1, W2)


def _tcD_body(parts_ref, dinv_ref, b2_ref, batch_ref, emb_ref,
              wah_ref, wae_ref, ba_ref, ws_ref, bs_ref, wo_ref, bo_ref,
              out_ref):
    dinv = dinv_ref[...]
    acc = parts_ref[0] + parts_ref[1]
    h2 = jnp.maximum(acc * dinv[:, None] + b2_ref[...][None, :], 0.0)

    P = lax.dot_general(emb_ref[...], wae_ref[...], (((1,), (1,)), ((), ())),
                        preferred_element_type=jnp.float32)
    batch = batch_ref[...]
    oh = (batch[:, None] == lax.broadcasted_iota(jnp.int32, (1, B), 1)
          ).astype(jnp.float32)
    embp = lax.dot_general(oh, P, (((1,), (0,)), ((), ())),
                           preferred_element_type=jnp.float32)
    ah = lax.dot_general(h2, wah_ref[...], (((1,), (1,)), ((), ())),
                         preferred_element_type=jnp.float32) + embp \
        + ba_ref[...][None, :]
    sc = jnp.where(ah >= 0, ah, 0.2 * ah)
    logits = jnp.sum(sc * ws_ref[...], axis=1, keepdims=True) \
        + bs_ref[...][None, :]
    m = jnp.max(logits)
    e = jnp.exp(logits - m)
    aw = e / jnp.sum(e)
    w = aw * h2
    S = lax.dot_general(oh, w, (((0,), (0,)), ((), ())),
                        preferred_element_type=jnp.float32)
    cnt = jnp.sum(oh, axis=0)
    num = jnp.sum(S * wo_ref[...][0][None, :], axis=1) + cnt * bo_ref[...]
    out_ref[...] = num / jnp.maximum(cnt, 1.0)


def _tcD(parts2, dinv, b2, batch, app_embedding, Wah, Wae, ba, Ws, bs,
         Wo, bo):
    return pl.pallas_call(
        _tcD_body,
        out_shape=jax.ShapeDtypeStruct((B,), jnp.float32),
    )(parts2, dinv, b2, batch, app_embedding, Wah, Wae, ba, Ws, bs,
      Wo, bo)


def kernel(x, edge_index, edge_attr, batch, app_embedding, W1, b1, W2, b2,
           Wa, ba, Ws, bs, Wo, bo):
    src = edge_index[0]
    dst = edge_index[1]
    ewi = lax.bitcast_convert_type(edge_attr, jnp.int32)
    ordr = np.concatenate([np.arange(16) + 32 * v for v in range(4)]
                          + [np.arange(16) + 32 * v + 16 for v in range(4)])
    W1s = W1[ordr]
    W2s = W2[ordr]
    degs, epk = _deg_kernel(src.reshape(NW, NCH0, K),
                            dst.reshape(NW, NCH0, K),
                            ewi.reshape(NW, NCH0, K),
                            jnp.asarray(_TAIL))
    ys1i, dinv = _tcA(x, W1s, degs)
    parts1 = _agg_kernel(epk, ys1i)
    ys2i = _tcC(parts1, dinv, b1, W2s)
    parts2 = _agg_kernel(epk, ys2i)
    Wah = Wa[:, :H]
    Wae = Wa[:, H:]
    return _tcD(parts2, dinv, b2, batch, app_embedding, Wah, Wae,
                ba, Ws, bs, Wo, bo)
